# baseline (speedup 1.0000x reference)
import functools

import jax
import jax.numpy as jnp
from jax import lax
from jax.experimental import pallas as pl
from jax.experimental.pallas import tpu as pltpu
from jax.experimental.pallas import tpu_sc as plsc

N_ROWS = 16384
N_CLASSES = 1000
C_PAD = 1024
BLOCK_R = 2048
GAMMA = 2.0
EPS = 1e-5

SC_CORES = 2
SC_SUBCORES = 16
LANES = 16
N_WORKERS = SC_CORES * SC_SUBCORES
ROWS_PER_WORKER = N_ROWS // N_WORKERS
ROWS_PER_SUBCORE = N_ROWS // SC_SUBCORES
RCHUNKS = ROWS_PER_WORKER // 128


def _sc_alpha_kernel(tg_hbm, init_hbm, xflat_hbm, a_hbm, xt_hbm,
                     th_v, tg_v, ones_v, hist_v, araw_v, a_v, xt_v, idx_v,
                     hist_sh, alpha_sh):
    c = lax.axis_index("c")
    s = lax.axis_index("s")
    wid = s * SC_CORES + c

    pltpu.sync_copy(tg_hbm.at[2 * s], th_v.at[pl.ds(0, RCHUNKS)])
    pltpu.sync_copy(tg_hbm.at[2 * s + 1], th_v.at[pl.ds(RCHUNKS, RCHUNKS)])
    pltpu.sync_copy(tg_hbm.at[wid], tg_v)
    for k in range(128 // LANES):
        ones_v[pl.ds(k * LANES, LANES)] = jnp.full((LANES,), 1.0, jnp.float32)

    base = wid * ROWS_PER_WORKER
    for ch in range(ROWS_PER_WORKER // LANES):
        j, o = ch // 8, (ch % 8) * LANES
        t16 = tg_v[j, pl.ds(o, LANES)]
        rows = jnp.full((LANES,), base + ch * LANES, jnp.int32) + lax.iota(
            jnp.int32, LANES)
        idx_v[j, pl.ds(o, LANES)] = rows * N_CLASSES + t16

    for j in range(RCHUNKS):
        pltpu.sync_copy(xflat_hbm.at[idx_v.at[j]], xt_v.at[j])
    pltpu.sync_copy(xt_v, xt_hbm.at[pl.ds(RCHUNKS * wid, RCHUNKS)])

    @pl.when(s == 0)
    def _():
        pltpu.sync_copy(init_hbm, hist_sh)

    plsc.subcore_barrier()

    for j in range(ROWS_PER_SUBCORE // 128):
        pltpu.sync_copy(ones_v, hist_sh.at[th_v.at[j]], add=True)

    plsc.subcore_barrier()
    pltpu.sync_copy(hist_sh, hist_v)

    acc = jnp.zeros((LANES,), jnp.float32)
    for j in range(C_PAD // LANES):
        h = hist_v[pl.ds(j * LANES, LANES)]
        ar = 1.0 / (h * (1.0 / N_ROWS) + EPS)
        araw_v[pl.ds(j * LANES, LANES)] = ar
        acc = acc + ar
    s2 = acc[0]
    for k in range(1, LANES):
        s2 = s2 + acc[k]
    inv_s2 = 1.0 / (jnp.full((LANES,), 1.0, jnp.float32) * s2)

    for j in range(C_PAD // LANES):
        araw_v[pl.ds(j * LANES, LANES)] = araw_v[pl.ds(j * LANES, LANES)] * inv_s2

    @pl.when(s == 0)
    def _():
        pltpu.sync_copy(araw_v, alpha_sh)

    plsc.subcore_barrier()
    for j in range(RCHUNKS):
        pltpu.sync_copy(alpha_sh.at[tg_v.at[j]], a_v.at[j])
    pltpu.sync_copy(a_v, a_hbm.at[pl.ds(RCHUNKS * wid, RCHUNKS)])


_sc_alpha = functools.partial(
    pl.kernel,
    out_type=(jax.ShapeDtypeStruct((128, 128), jnp.float32),
              jax.ShapeDtypeStruct((128, 128), jnp.float32)),
    mesh=plsc.VectorSubcoreMesh(core_axis_name="c", subcore_axis_name="s"),
    scratch_types=[
        pltpu.VMEM((2 * RCHUNKS, 128), jnp.int32),
        pltpu.VMEM((RCHUNKS, 128), jnp.int32),
        pltpu.VMEM((128,), jnp.float32),
        pltpu.VMEM((C_PAD,), jnp.float32),
        pltpu.VMEM((C_PAD,), jnp.float32),
        pltpu.VMEM((RCHUNKS, 128), jnp.float32),
        pltpu.VMEM((RCHUNKS, 128), jnp.float32),
        pltpu.VMEM((RCHUNKS, 128), jnp.int32),
        pltpu.VMEM_SHARED((C_PAD,), jnp.float32),
        pltpu.VMEM_SHARED((C_PAD,), jnp.float32),
    ],
)(_sc_alpha_kernel)


def _stream_kernel(x_ref, lse_ref):
    x = x_ref[...]
    m = jnp.max(x, axis=1, keepdims=True)
    e = jnp.exp(x - m)
    ones = jnp.ones((N_CLASSES, 1), jnp.float32)
    s = lax.dot_general(e, ones, (((1,), (0,)), ((), ())),
                        preferred_element_type=jnp.float32)
    lse_ref[...] = m + jnp.log(s)


def _combine_kernel(lse_ref, a_ref, xt_ref, out_ref):
    lse = lse_ref[...].reshape(128, 128)
    a = a_ref[...]
    xt = xt_ref[...]
    nll = lse - xt
    ce = a * nll
    pt = jnp.exp(-ce)
    loss = (1.0 - pt) ** GAMMA * ce
    out_ref[...] = jnp.sum(loss).reshape(1, 1) / N_ROWS


def kernel(inputs, targets):
    t32 = targets.astype(jnp.int32)
    tg = t32.reshape(N_WORKERS, RCHUNKS, 128)
    init_hist = jnp.concatenate(
        [jnp.zeros((N_CLASSES,), jnp.float32),
         jnp.full((C_PAD - N_CLASSES,), 1e30, jnp.float32)])

    xflat = lax.optimization_barrier(inputs.reshape(N_ROWS * N_CLASSES))
    x2d = xflat.reshape(N_ROWS, N_CLASSES)

    a2, xt2 = _sc_alpha(tg, init_hist, xflat)

    nb = N_ROWS // BLOCK_R
    lse = pl.pallas_call(
        _stream_kernel,
        grid=(nb,),
        in_specs=[pl.BlockSpec((BLOCK_R, N_CLASSES), lambda i: (i, 0))],
        out_specs=pl.BlockSpec((BLOCK_R, 1), lambda i: (i, 0)),
        out_shape=jax.ShapeDtypeStruct((N_ROWS, 1), jnp.float32),
        compiler_params=pltpu.CompilerParams(
            dimension_semantics=("parallel",)),
    )(x2d)

    out = pl.pallas_call(
        _combine_kernel,
        in_specs=[
            pl.BlockSpec((N_ROWS, 1), lambda: (0, 0)),
            pl.BlockSpec((128, 128), lambda: (0, 0)),
            pl.BlockSpec((128, 128), lambda: (0, 0)),
        ],
        out_specs=pl.BlockSpec((1, 1), lambda: (0, 0)),
        out_shape=jax.ShapeDtypeStruct((1, 1), jnp.float32),
    )(lse, a2, xt2)

    return out[0, 0]


# speedup vs baseline: 2.1418x; 2.1418x over previous
import functools

import jax
import jax.numpy as jnp
from jax import lax
from jax.experimental import pallas as pl
from jax.experimental.pallas import tpu as pltpu
from jax.experimental.pallas import tpu_sc as plsc

N_ROWS = 16384
N_CLASSES = 1000
C_PAD = 1024
BLOCK_R = 2048
GAMMA = 2.0
EPS = 1e-5

SC_CORES = 2
SC_SUBCORES = 16
LANES = 16
N_WORKERS = SC_CORES * SC_SUBCORES
ROWS_PER_WORKER = N_ROWS // N_WORKERS
ROWS_PER_SUBCORE = N_ROWS // SC_SUBCORES


def _sc_alpha_kernel(tg_hbm, init_hbm, out_hbm,
                     th_v, tg_v, ones_v, hist_v, araw_v, a_v, hist_sh, alpha_sh):
    c = lax.axis_index("c")
    s = lax.axis_index("s")
    wid = s * SC_CORES + c

    pltpu.sync_copy(tg_hbm.at[2 * s], th_v.at[pl.ds(0, 4)])
    pltpu.sync_copy(tg_hbm.at[2 * s + 1], th_v.at[pl.ds(4, 4)])
    pltpu.sync_copy(tg_hbm.at[wid], tg_v)
    for k in range(128 // LANES):
        ones_v[pl.ds(k * LANES, LANES)] = jnp.full((LANES,), 1.0, jnp.float32)

    @pl.when(s == 0)
    def _():
        pltpu.sync_copy(init_hbm, hist_sh)

    plsc.subcore_barrier()

    for j in range(ROWS_PER_SUBCORE // 128):
        pltpu.sync_copy(ones_v, hist_sh.at[th_v.at[j]], add=True)

    plsc.subcore_barrier()
    pltpu.sync_copy(hist_sh, hist_v)

    acc = jnp.zeros((LANES,), jnp.float32)
    for j in range(C_PAD // LANES):
        h = hist_v[pl.ds(j * LANES, LANES)]
        ar = 1.0 / (h * (1.0 / N_ROWS) + EPS)
        araw_v[pl.ds(j * LANES, LANES)] = ar
        acc = acc + ar
    s2 = acc[0]
    for k in range(1, LANES):
        s2 = s2 + acc[k]
    inv_s2 = 1.0 / (jnp.full((LANES,), 1.0, jnp.float32) * s2)

    for j in range(C_PAD // LANES):
        araw_v[pl.ds(j * LANES, LANES)] = araw_v[pl.ds(j * LANES, LANES)] * inv_s2

    @pl.when(s == 0)
    def _():
        pltpu.sync_copy(araw_v, alpha_sh)

    plsc.subcore_barrier()
    for j in range(ROWS_PER_WORKER // 128):
        pltpu.sync_copy(alpha_sh.at[tg_v.at[j]], a_v.at[j])

    pltpu.sync_copy(a_v, out_hbm.at[pl.ds(4 * wid, 4)])


_sc_alpha = functools.partial(
    pl.kernel,
    out_type=jax.ShapeDtypeStruct((128, 128), jnp.float32),
    mesh=plsc.VectorSubcoreMesh(core_axis_name="c", subcore_axis_name="s"),
    scratch_types=[
        pltpu.VMEM((ROWS_PER_SUBCORE // 128, 128), jnp.int32),
        pltpu.VMEM((ROWS_PER_WORKER // 128, 128), jnp.int32),
        pltpu.VMEM((128,), jnp.float32),
        pltpu.VMEM((C_PAD,), jnp.float32),
        pltpu.VMEM((C_PAD,), jnp.float32),
        pltpu.VMEM((ROWS_PER_WORKER // 128, 128), jnp.float32),
        pltpu.VMEM_SHARED((C_PAD,), jnp.float32),
        pltpu.VMEM_SHARED((C_PAD,), jnp.float32),
    ],
)(_sc_alpha_kernel)


def _stream_kernel(x_ref, t_ref, nll_ref):
    x = x_ref[...]
    t = t_ref[...]
    m = jnp.max(x, axis=1, keepdims=True)
    e = jnp.exp(x - m)
    iota = lax.broadcasted_iota(jnp.int32, (BLOCK_R, N_CLASSES), 1)
    w = jnp.where(iota == t, x, 0.0)
    ones = jnp.ones((N_CLASSES, 1), jnp.float32)
    s = lax.dot_general(e, ones, (((1,), (0,)), ((), ())),
                        preferred_element_type=jnp.float32)
    xt = lax.dot_general(w, ones, (((1,), (0,)), ((), ())),
                         preferred_element_type=jnp.float32)
    nll_ref[...] = m + jnp.log(s) - xt


def _combine_kernel(nll_ref, a_ref, out_ref):
    nll = nll_ref[...]
    a = a_ref[...]
    ce = a * nll
    pt = jnp.exp(-ce)
    loss = (1.0 - pt) ** GAMMA * ce
    out_ref[...] = jnp.sum(loss).reshape(1, 1) / N_ROWS


def kernel(inputs, targets):
    t32 = targets.astype(jnp.int32)
    tg = t32.reshape(N_WORKERS, ROWS_PER_WORKER // 128, 128)
    init_hist = jnp.concatenate(
        [jnp.zeros((N_CLASSES,), jnp.float32),
         jnp.full((C_PAD - N_CLASSES,), 1e30, jnp.float32)])

    a = _sc_alpha(tg, init_hist)

    nb = N_ROWS // BLOCK_R
    nll = pl.pallas_call(
        _stream_kernel,
        grid=(nb,),
        in_specs=[
            pl.BlockSpec((BLOCK_R, N_CLASSES), lambda i: (i, 0)),
            pl.BlockSpec((BLOCK_R, 1), lambda i: (i, 0)),
        ],
        out_specs=pl.BlockSpec((BLOCK_R, 1), lambda i: (i, 0)),
        out_shape=jax.ShapeDtypeStruct((N_ROWS, 1), jnp.float32),
        compiler_params=pltpu.CompilerParams(
            dimension_semantics=("parallel",)),
    )(inputs, t32.reshape(N_ROWS, 1))

    out = pl.pallas_call(
        _combine_kernel,
        in_specs=[
            pl.BlockSpec((128, 128), lambda: (0, 0)),
            pl.BlockSpec((128, 128), lambda: (0, 0)),
        ],
        out_specs=pl.BlockSpec((1, 1), lambda: (0, 0)),
        out_shape=jax.ShapeDtypeStruct((1, 1), jnp.float32),
    )(nll.reshape(128, 128), a)

    return out[0, 0]


# speedup vs baseline: 2.1784x; 1.0171x over previous
import functools

import jax
import jax.numpy as jnp
from jax import lax
from jax.experimental import pallas as pl
from jax.experimental.pallas import tpu as pltpu
from jax.experimental.pallas import tpu_sc as plsc

N_ROWS = 16384
N_CLASSES = 1000
C_PAD = 1024
BLOCK_R = 2048
GAMMA = 2.0
EPS = 1e-5

SC_CORES = 2
SC_SUBCORES = 16
LANES = 16
N_WORKERS = SC_CORES * SC_SUBCORES
ROWS_PER_WORKER = N_ROWS // N_WORKERS
ROWS_PER_SUBCORE = N_ROWS // SC_SUBCORES


def _sc_alpha_kernel(tg_hbm, init_hbm, out_hbm,
                     th_v, tg_v, ones_v, hist_v, araw_v, a_v, hist_sh, alpha_sh):
    c = lax.axis_index("c")
    s = lax.axis_index("s")
    wid = s * SC_CORES + c

    pltpu.sync_copy(tg_hbm.at[2 * s], th_v.at[pl.ds(0, 4)])
    pltpu.sync_copy(tg_hbm.at[2 * s + 1], th_v.at[pl.ds(4, 4)])
    pltpu.sync_copy(tg_hbm.at[wid], tg_v)
    for k in range(128 // LANES):
        ones_v[pl.ds(k * LANES, LANES)] = jnp.full((LANES,), 1.0, jnp.float32)

    @pl.when(s == 0)
    def _():
        pltpu.sync_copy(init_hbm, hist_sh)

    plsc.subcore_barrier()

    for j in range(ROWS_PER_SUBCORE // 128):
        pltpu.sync_copy(ones_v, hist_sh.at[th_v.at[j]], add=True)

    plsc.subcore_barrier()
    pltpu.sync_copy(hist_sh, hist_v)

    acc = jnp.zeros((LANES,), jnp.float32)
    for j in range(C_PAD // LANES):
        h = hist_v[pl.ds(j * LANES, LANES)]
        ar = 1.0 / (h * (1.0 / N_ROWS) + EPS)
        araw_v[pl.ds(j * LANES, LANES)] = ar
        acc = acc + ar
    s2 = acc[0]
    for k in range(1, LANES):
        s2 = s2 + acc[k]
    inv_s2 = 1.0 / (jnp.full((LANES,), 1.0, jnp.float32) * s2)

    for j in range(C_PAD // LANES):
        araw_v[pl.ds(j * LANES, LANES)] = araw_v[pl.ds(j * LANES, LANES)] * inv_s2

    @pl.when(s == 0)
    def _():
        pltpu.sync_copy(araw_v, alpha_sh)

    plsc.subcore_barrier()
    for j in range(ROWS_PER_WORKER // 128):
        pltpu.sync_copy(alpha_sh.at[tg_v.at[j]], a_v.at[pl.ds(j * 128, 128)])

    pltpu.sync_copy(a_v, out_hbm.at[pl.ds(wid * ROWS_PER_WORKER,
                                          ROWS_PER_WORKER)])


_sc_alpha = functools.partial(
    pl.kernel,
    out_type=jax.ShapeDtypeStruct((N_ROWS,), jnp.float32),
    mesh=plsc.VectorSubcoreMesh(core_axis_name="c", subcore_axis_name="s"),
    scratch_types=[
        pltpu.VMEM((ROWS_PER_SUBCORE // 128, 128), jnp.int32),
        pltpu.VMEM((ROWS_PER_WORKER // 128, 128), jnp.int32),
        pltpu.VMEM((128,), jnp.float32),
        pltpu.VMEM((C_PAD,), jnp.float32),
        pltpu.VMEM((C_PAD,), jnp.float32),
        pltpu.VMEM((ROWS_PER_WORKER,), jnp.float32),
        pltpu.VMEM_SHARED((C_PAD,), jnp.float32),
        pltpu.VMEM_SHARED((C_PAD,), jnp.float32),
    ],
)(_sc_alpha_kernel)


def _stream_kernel(x_ref, t_ref, nll_ref):
    x = x_ref[...]
    t = t_ref[...]
    m = jnp.max(x, axis=1, keepdims=True)
    e = jnp.exp(x - m)
    iota = lax.broadcasted_iota(jnp.int32, (BLOCK_R, N_CLASSES), 1)
    w = jnp.where(iota == t, x, 0.0)
    ones = jnp.ones((N_CLASSES, 1), jnp.float32)
    s = lax.dot_general(e, ones, (((1,), (0,)), ((), ())),
                        preferred_element_type=jnp.float32)
    xt = lax.dot_general(w, ones, (((1,), (0,)), ((), ())),
                         preferred_element_type=jnp.float32)
    nll_ref[...] = m + jnp.log(s) - xt


def _combine_kernel(nll_ref, a_ref, out_ref):
    nll = nll_ref[...]
    a = a_ref[...]
    ce = a * nll
    pt = jnp.exp(-ce)
    loss = (1.0 - pt) ** GAMMA * ce
    out_ref[...] = jnp.sum(loss).reshape(1, 1) / N_ROWS


def kernel(inputs, targets):
    t32 = targets.astype(jnp.int32)
    tg = t32.reshape(N_WORKERS, ROWS_PER_WORKER // 128, 128)
    init_hist = jnp.concatenate(
        [jnp.zeros((N_CLASSES,), jnp.float32),
         jnp.full((C_PAD - N_CLASSES,), 1e30, jnp.float32)])

    a = _sc_alpha(tg, init_hist)

    nb = N_ROWS // BLOCK_R
    nll = pl.pallas_call(
        _stream_kernel,
        grid=(nb,),
        in_specs=[
            pl.BlockSpec((BLOCK_R, N_CLASSES), lambda i: (i, 0)),
            pl.BlockSpec((BLOCK_R, 1), lambda i: (i, 0)),
        ],
        out_specs=pl.BlockSpec((BLOCK_R, 1), lambda i: (i, 0)),
        out_shape=jax.ShapeDtypeStruct((N_ROWS, 1), jnp.float32),
        compiler_params=pltpu.CompilerParams(
            dimension_semantics=("parallel",)),
    )(inputs, t32.reshape(N_ROWS, 1))

    out = pl.pallas_call(
        _combine_kernel,
        in_specs=[
            pl.BlockSpec((128, 128), lambda: (0, 0)),
            pl.BlockSpec((128, 128), lambda: (0, 0)),
        ],
        out_specs=pl.BlockSpec((1, 1), lambda: (0, 0)),
        out_shape=jax.ShapeDtypeStruct((1, 1), jnp.float32),
    )(nll.reshape(128, 128), a.reshape(128, 128))

    return out[0, 0]
